# baseline (device time: 114078 ns/iter reference)
import jax
import jax.numpy as jnp
from jax import lax
from jax.experimental import pallas as pl
from jax.experimental.pallas import tpu as pltpu

N_DEV = 8
_XOR_MASKS = (1, 3, 4)


def _layer(x, Win, Wout, collective_id):
    b, d = x.shape
    d_in, h_per = Win.shape
    nt = 8
    t = h_per // nt

    def body(x_ref, win_ref, wout_ref, out_ref,
             xbf_ref, acc_ref, comm_ref, recv_ref, send_sems, recv_sems):
        j = pl.program_id(0)
        my = lax.axis_index("i")

        @pl.when(j == 0)
        def _entry():
            barrier = pltpu.get_barrier_semaphore()
            for m in _XOR_MASKS:
                pl.semaphore_signal(
                    barrier, inc=1,
                    device_id=(my ^ m,),
                    device_id_type=pl.DeviceIdType.MESH,
                )
            pl.semaphore_wait(barrier, len(_XOR_MASKS))
            xbf_ref[...] = x_ref[...].astype(jnp.bfloat16)
            acc_ref[...] = jnp.zeros((b, d), jnp.float32)

        wi = win_ref[...].astype(jnp.bfloat16)
        h = lax.dot_general(
            xbf_ref[...], wi, (((1,), (0,)), ((), ())),
            preferred_element_type=jnp.float32,
        )
        h = jnp.maximum(h, 0.0).astype(jnp.bfloat16)
        wo = wout_ref[...].astype(jnp.bfloat16)
        acc_ref[...] += lax.dot_general(
            h, wo, (((1,), (0,)), ((), ())),
            preferred_element_type=jnp.float32,
        )

        @pl.when(j == nt - 1)
        def _allreduce():
            comm_ref[0, :, :] = acc_ref[...].astype(jnp.bfloat16)
            for r, m in enumerate(_XOR_MASKS):
                rdma = pltpu.make_async_remote_copy(
                    src_ref=comm_ref.at[r],
                    dst_ref=recv_ref.at[r],
                    send_sem=send_sems.at[r],
                    recv_sem=recv_sems.at[r],
                    device_id=(my ^ m,),
                    device_id_type=pl.DeviceIdType.MESH,
                )
                rdma.start()
                rdma.wait()
                comm_ref[r + 1, :, :] = (
                    comm_ref[r].astype(jnp.float32)
                    + recv_ref[r].astype(jnp.float32)
                ).astype(jnp.bfloat16)
            out_ref[...] = comm_ref[len(_XOR_MASKS)].astype(jnp.float32)

    return pl.pallas_call(
        body,
        grid=(nt,),
        out_shape=jax.ShapeDtypeStruct((b, d), jnp.float32),
        in_specs=[
            pl.BlockSpec((b, d), lambda j: (0, 0)),
            pl.BlockSpec((d_in, t), lambda j: (0, j)),
            pl.BlockSpec((t, d), lambda j: (j, 0)),
        ],
        out_specs=pl.BlockSpec((b, d), lambda j: (0, 0)),
        scratch_shapes=[
            pltpu.VMEM((b, d), jnp.bfloat16),
            pltpu.VMEM((b, d), jnp.float32),
            pltpu.VMEM((4, b, d), jnp.bfloat16),
            pltpu.VMEM((3, b, d), jnp.bfloat16),
            pltpu.SemaphoreType.DMA((3,)),
            pltpu.SemaphoreType.DMA((3,)),
        ],
        compiler_params=pltpu.CompilerParams(
            dimension_semantics=("arbitrary",),
            collective_id=collective_id,
        ),
    )(x, Win, Wout)


def kernel(x, Win0, Wout0, Win1, Wout1, Win2, Wout2):
    x = _layer(x, Win0, Wout0, 0)
    x = _layer(x, Win1, Wout1, 1)
    x = _layer(x, Win2, Wout2, 2)
    return x
